# Initial kernel scaffold; baseline (speedup 1.0000x reference)
import jax
import jax.numpy as jnp
from jax.experimental import pallas as pl


def kernel(similarity_matrix, scores, threshold):
    raise NotImplementedError("write your pallas kernel here")


# speedup vs baseline: 35.2788x; 35.2788x over previous
import functools

import jax
import jax.numpy as jnp
from jax import lax
from jax.experimental import pallas as pl
from jax.experimental.pallas import tpu as pltpu
from jax.experimental.pallas import tpu_sc as plsc

L = 16
NC = 2
NS = 16
NW = NC * NS


def _argmax(scores_v, n_sl):
    iota = lax.iota(jnp.int32, L)
    best_v = scores_v[pl.ds(0, L)]
    best_i = iota
    for j in range(1, n_sl):
        v = scores_v[pl.ds(j * L, L)]
        upd = v > best_v
        best_v = jnp.where(upd, v, best_v)
        best_i = jnp.where(upd, iota + j * L, best_i)
    m = plsc.cummax(best_v)[L - 1]
    cand = jnp.where(best_v == m, best_i, jnp.int32(n_sl * L))
    return m, -plsc.cummax(-cand)[L - 1]


def _nms_body(n, n_sl, groups_per_worker,
              sim_rows_hbm, scores_hbm, thr_hbm, keep_hbm, ret_hbm,
              scores_v, ret_v, keep_v, row_v, thr_v):
    wid = lax.axis_index("s") * NC + lax.axis_index("c")
    iota = lax.iota(jnp.int32, L)
    lane0 = iota == 0
    neg1 = jnp.full((L,), -1, jnp.int32)

    for k in range(groups_per_worker):
        g = wid * groups_per_worker + k
        pltpu.sync_copy(scores_hbm.at[g], scores_v)
        pltpu.sync_copy(scores_hbm.at[g], ret_v)
        pltpu.sync_copy(thr_hbm.at[g], thr_v)
        for j in range(n_sl):
            keep_v[pl.ds(j * L, L)] = neg1
        thr_vec = thr_v[...]
        row_base = g * n

        m0, idx0 = _argmax(scores_v, n_sl)

        def cond(c):
            i, m, idx = c
            return (m != 0.0) & (i < n)

        def body(c):
            i, m, idx = c
            plsc.store_scatter(keep_v, [jnp.full((L,), i, jnp.int32)],
                               jnp.full((L,), idx, jnp.int32), mask=lane0)
            plsc.store_scatter(ret_v, [jnp.full((L,), idx, jnp.int32)],
                               jnp.full((L,), 1000.0 - i.astype(jnp.float32),
                                        jnp.float32), mask=lane0)
            pltpu.sync_copy(sim_rows_hbm.at[row_base + idx], row_v)
            for j in range(n_sl):
                sl = pl.ds(j * L, L)
                s = scores_v[sl]
                r = row_v[sl]
                dead = (r > thr_vec) | (iota + j * L == idx)
                scores_v[sl] = jnp.where(dead, 0.0, s)
            m2, idx2 = _argmax(scores_v, n_sl)
            return (i + 1, m2, idx2)

        lax.while_loop(cond, body, (jnp.int32(0), m0, idx0))

        pltpu.sync_copy(keep_v, keep_hbm.at[g])
        pltpu.sync_copy(ret_v, ret_hbm.at[g])


@functools.partial(jax.jit, static_argnums=(3, 4, 5))
def _sc_nms(sim_rows, scores_t, thr, bg, n, n_sl):
    groups_per_worker = bg // NW
    mesh = plsc.VectorSubcoreMesh(core_axis_name="c", subcore_axis_name="s",
                                  num_cores=NC, num_subcores=NS)
    body = functools.partial(_nms_body, n, n_sl, groups_per_worker)
    return pl.kernel(
        body,
        out_type=[jax.ShapeDtypeStruct((bg, n), jnp.int32),
                  jax.ShapeDtypeStruct((bg, n), jnp.float32)],
        mesh=mesh,
        compiler_params=pltpu.CompilerParams(needs_layout_passes=False),
        scratch_types=[
            pltpu.VMEM((n,), jnp.float32),
            pltpu.VMEM((n,), jnp.float32),
            pltpu.VMEM((n,), jnp.int32),
            pltpu.VMEM((n,), jnp.float32),
            pltpu.VMEM((L,), jnp.float32),
        ],
    )(sim_rows, scores_t, thr)


def kernel(similarity_matrix, scores, threshold):
    B, G, N, _ = similarity_matrix.shape
    bg = B * G
    assert bg % NW == 0 and N % L == 0
    sim_rows = similarity_matrix.reshape(bg * N, N)
    scores_t = jnp.transpose(scores, (0, 2, 1)).reshape(bg, N)
    thr = jnp.broadcast_to(threshold[:, None, None], (B, G, L)).reshape(bg, L)
    keep_flat, ret_flat = _sc_nms(sim_rows, scores_t, thr, bg, N, N // L)
    keep = keep_flat.reshape(B, G, N)
    ret = jnp.transpose(ret_flat.reshape(B, G, N), (0, 2, 1))
    return keep, ret


# speedup vs baseline: 66.0189x; 1.8713x over previous
import functools

import jax
import jax.numpy as jnp
from jax import lax
from jax.experimental import pallas as pl
from jax.experimental.pallas import tpu as pltpu
from jax.experimental.pallas import tpu_sc as plsc

L = 16
NC = 2
NS = 16
NW = NC * NS
GPW = 3


def _lane_reduce(best_v, best_i, n_sl):
    m = plsc.cummax(best_v)[L - 1]
    cand = jnp.where(best_v == m, best_i, jnp.int32(n_sl * L))
    return m, -plsc.cummax(-cand)[L - 1]


def _argmax_fresh(scores_v, n_sl):
    iota = lax.iota(jnp.int32, L)
    best_v = scores_v[pl.ds(0, L)]
    best_i = iota
    for j in range(1, n_sl):
        v = scores_v[pl.ds(j * L, L)]
        upd = v > best_v
        best_v = jnp.where(upd, v, best_v)
        best_i = jnp.where(upd, iota + j * L, best_i)
    return _lane_reduce(best_v, best_i, n_sl)


def _nms_body(n, n_sl,
              sim_rows_hbm, scores_hbm, thr_hbm, keep_hbm, ret_hbm,
              *scr):
    scores_v = scr[0:3]
    ret_v = scr[3:6]
    keep_v = scr[6:9]
    row_v = scr[9:12]
    thr_v = scr[12:15]
    sems = scr[15:18]
    wid = lax.axis_index("s") * NC + lax.axis_index("c")
    iota = lax.iota(jnp.int32, L)
    lane0 = iota == 0
    neg1 = jnp.full((L,), -1, jnp.int32)

    gs = [wid + k * NW for k in range(GPW)]

    def record(k, i, idx):
        plsc.store_scatter(keep_v[k], [jnp.full((L,), i, jnp.int32)],
                           jnp.full((L,), idx, jnp.int32), mask=lane0)
        plsc.store_scatter(ret_v[k], [jnp.full((L,), idx, jnp.int32)],
                           jnp.full((L,), 1000.0 - i.astype(jnp.float32),
                                    jnp.float32), mask=lane0)

    def start_row_dma(k, idx):
        pltpu.make_async_copy(sim_rows_hbm.at[gs[k] * n + idx],
                              row_v[k], sems[k]).start()

    def wait_row_dma(k):
        pltpu.make_async_copy(sim_rows_hbm.at[gs[k] * n], row_v[k],
                              sems[k]).wait()

    def suppress_argmax(k, idx):
        thr_vec = thr_v[k][...]
        best_v = jnp.full((L,), 0.0, jnp.float32)
        best_i = jnp.zeros((L,), jnp.int32)
        first = True
        for j in range(n_sl):
            sl = pl.ds(j * L, L)
            lanes = iota + j * L
            dead = (row_v[k][sl] > thr_vec) | (lanes == idx)
            s = jnp.where(dead, 0.0, scores_v[k][sl])
            scores_v[k][sl] = s
            if first:
                best_v, best_i, first = s, lanes, False
            else:
                upd = s > best_v
                best_v = jnp.where(upd, s, best_v)
                best_i = jnp.where(upd, lanes, best_i)
        return _lane_reduce(best_v, best_i, n_sl)

    state0 = []
    for k in range(GPW):
        g = gs[k]
        pltpu.sync_copy(scores_hbm.at[g], scores_v[k])
        pltpu.sync_copy(scores_hbm.at[g], ret_v[k])
        pltpu.sync_copy(thr_hbm.at[g], thr_v[k])
        for j in range(n_sl):
            keep_v[k][pl.ds(j * L, L)] = neg1
        m0, idx0 = _argmax_fresh(scores_v[k], n_sl)
        act0 = m0 != 0.0

        @pl.when(act0)
        def _():
            record(k, jnp.int32(0), idx0)

        start_row_dma(k, idx0)
        state0 += [act0, jnp.int32(1), idx0]

    def cond(c):
        return c[0] | c[3] | c[6]

    def body(c):
        out = list(c)
        for k in range(GPW):
            i, idx = c[3 * k + 1], c[3 * k + 2]
            wait_row_dma(k)
            m2, idx2 = suppress_argmax(k, idx)
            rec = (m2 != 0.0) & (i < n)

            @pl.when(rec)
            def _():
                record(k, i, idx2)

            start_row_dma(k, idx2)
            out[3 * k], out[3 * k + 1], out[3 * k + 2] = rec, i + 1, idx2
        return tuple(out)

    lax.while_loop(cond, body, tuple(state0))

    for k in range(GPW):
        wait_row_dma(k)
        pltpu.sync_copy(keep_v[k], keep_hbm.at[gs[k]])
        pltpu.sync_copy(ret_v[k], ret_hbm.at[gs[k]])


@functools.partial(jax.jit, static_argnums=(3, 4, 5))
def _sc_nms(sim_rows, scores_t, thr, bg, n, n_sl):
    mesh = plsc.VectorSubcoreMesh(core_axis_name="c", subcore_axis_name="s",
                                  num_cores=NC, num_subcores=NS)
    body = functools.partial(_nms_body, n, n_sl)
    return pl.kernel(
        body,
        out_type=[jax.ShapeDtypeStruct((bg, n), jnp.int32),
                  jax.ShapeDtypeStruct((bg, n), jnp.float32)],
        mesh=mesh,
        compiler_params=pltpu.CompilerParams(needs_layout_passes=False),
        scratch_types=(
            [pltpu.VMEM((n,), jnp.float32)] * GPW
            + [pltpu.VMEM((n,), jnp.float32)] * GPW
            + [pltpu.VMEM((n,), jnp.int32)] * GPW
            + [pltpu.VMEM((n,), jnp.float32)] * GPW
            + [pltpu.VMEM((L,), jnp.float32)] * GPW
            + [pltpu.SemaphoreType.DMA] * GPW
        ),
    )(sim_rows, scores_t, thr)


def kernel(similarity_matrix, scores, threshold):
    B, G, N, _ = similarity_matrix.shape
    bg = B * G
    assert bg == NW * GPW and N % L == 0
    sim_rows = similarity_matrix.reshape(bg * N, N)
    scores_t = jnp.transpose(scores, (0, 2, 1)).reshape(bg, N)
    thr = jnp.broadcast_to(threshold[:, None, None], (B, G, L)).reshape(bg, L)
    keep_flat, ret_flat = _sc_nms(sim_rows, scores_t, thr, bg, N, N // L)
    keep = keep_flat.reshape(B, G, N)
    ret = jnp.transpose(ret_flat.reshape(B, G, N), (0, 2, 1))
    return keep, ret


# speedup vs baseline: 71.7991x; 1.0876x over previous
import functools

import jax
import jax.numpy as jnp
from jax import lax
from jax.experimental import pallas as pl
from jax.experimental.pallas import tpu as pltpu
from jax.experimental.pallas import tpu_sc as plsc

L = 16
NC = 2
NS = 16
NW = NC * NS
GPW = 3


def _min_index_of(value_v, best_v, best_i, big):
    cand = jnp.where(best_v == value_v, best_i, jnp.int32(big))
    return -plsc.cummax(-cand)[L - 1]


def _lane_top2(best_v, best_i, sec_v, sec_i, big):
    m1 = plsc.cummax(best_v)[L - 1]
    i1 = _min_index_of(m1, best_v, best_i, big)
    is_w = best_i == jnp.full((L,), i1, jnp.int32)
    scv = jnp.where(is_w, sec_v, best_v)
    sci = jnp.where(is_w, sec_i, best_i)
    m2 = plsc.cummax(scv)[L - 1]
    i2 = _min_index_of(m2, scv, sci, big)
    return m1, i1, m2, i2


def _nms_body(n, n_sl,
              sim_rows_hbm, scores_hbm, thr_hbm, keep_hbm, ret_hbm,
              *scr):
    scores_v = scr[0:3]
    ret_v = scr[3:6]
    keep_v = scr[6:9]
    rowa_v = scr[9:12]
    rowb_v = scr[12:15]
    thr_v = scr[15:18]
    sema = scr[18:21]
    semb = scr[21:24]
    wid = lax.axis_index("s") * NC + lax.axis_index("c")
    iota = lax.iota(jnp.int32, L)
    lane0 = iota == 0
    neg1 = jnp.full((L,), -1, jnp.int32)
    big = n_sl * L

    gs = [wid + k * NW for k in range(GPW)]

    def record(k, i, idx):
        plsc.store_scatter(keep_v[k], [jnp.full((L,), i, jnp.int32)],
                           jnp.full((L,), idx, jnp.int32), mask=lane0)
        plsc.store_scatter(ret_v[k], [jnp.full((L,), idx, jnp.int32)],
                           jnp.full((L,), 1000.0 - i.astype(jnp.float32),
                                    jnp.float32), mask=lane0)

    def start_dma(k, idx, buf, sem):
        pltpu.make_async_copy(sim_rows_hbm.at[gs[k] * n + idx],
                              buf[k], sem[k]).start()

    def wait_dma(k, buf, sem):
        pltpu.make_async_copy(sim_rows_hbm.at[gs[k] * n], buf[k],
                              sem[k]).wait()

    def top2_sweep(k, dead_of):
        best_v = jnp.full((L,), 0.0, jnp.float32)
        sec_v = jnp.full((L,), 0.0, jnp.float32)
        best_i = jnp.zeros((L,), jnp.int32)
        sec_i = jnp.zeros((L,), jnp.int32)
        first = True
        for j in range(n_sl):
            sl = pl.ds(j * L, L)
            lanes = iota + j * L
            s = scores_v[k][sl]
            dead = dead_of(k, sl)
            if dead is not None:
                s = jnp.where(dead, 0.0, s)
                scores_v[k][sl] = s
            if first:
                best_v, best_i, first = s, lanes, False
            else:
                upd1 = s > best_v
                upd2 = s > sec_v
                nsec_v = jnp.where(upd1, best_v, jnp.where(upd2, s, sec_v))
                nsec_i = jnp.where(upd1, best_i, jnp.where(upd2, lanes, sec_i))
                best_v = jnp.where(upd1, s, best_v)
                best_i = jnp.where(upd1, lanes, best_i)
                sec_v, sec_i = nsec_v, nsec_i
        return _lane_top2(best_v, best_i, sec_v, sec_i, big)

    state0 = []
    for k in range(GPW):
        g = gs[k]
        pltpu.sync_copy(scores_hbm.at[g], scores_v[k])
        pltpu.sync_copy(scores_hbm.at[g], ret_v[k])
        pltpu.sync_copy(thr_hbm.at[g], thr_v[k])
        for j in range(n_sl):
            keep_v[k][pl.ds(j * L, L)] = neg1
        m1, c1, m2, c2 = top2_sweep(k, lambda k_, sl: None)
        act0 = m1 != 0.0

        @pl.when(act0)
        def _():
            record(k, jnp.int32(0), c1)

        start_dma(k, c1, rowa_v, sema)
        start_dma(k, c2, rowb_v, semb)
        state0 += [act0, jnp.int32(1), c1, c2, m2]

    def cond(c):
        return c[0] | c[5] | c[10]

    def body(c):
        out = list(c)
        for k in range(GPW):
            i, c1, c2, m2val = (c[5 * k + 1], c[5 * k + 2],
                                c[5 * k + 3], c[5 * k + 4])
            thr_vec = thr_v[k][...]
            thr_s = thr_vec[0]
            wait_dma(k, rowa_v, sema)
            wait_dma(k, rowb_v, semb)
            r1c2 = plsc.load_gather(rowa_v[k],
                                    [jnp.full((L,), c2, jnp.int32)])[0]
            hit = (r1c2 <= thr_s) & (m2val != 0.0)
            hitv = jnp.full((L,), hit, jnp.bool_)

            @pl.when(hit & (i < n))
            def _():
                record(k, i, c2)

            i1 = i + hit.astype(jnp.int32)
            plsc.store_scatter(scores_v[k], [jnp.full((L,), c1, jnp.int32)],
                               jnp.zeros((L,), jnp.float32), mask=lane0)
            plsc.store_scatter(scores_v[k], [jnp.full((L,), c2, jnp.int32)],
                               jnp.zeros((L,), jnp.float32),
                               mask=lane0 & hitv)

            def dead_of(k_, sl):
                d1 = rowa_v[k_][sl] > thr_vec
                d2 = (rowb_v[k_][sl] > thr_vec) & hitv
                return d1 | d2

            m1n, nc1, m2n, nc2 = top2_sweep(k, dead_of)
            act = (m1n != 0.0) & (i1 < n)

            @pl.when(act)
            def _():
                record(k, i1, nc1)

            start_dma(k, nc1, rowa_v, sema)
            start_dma(k, nc2, rowb_v, semb)
            out[5 * k:5 * k + 5] = [act, i1 + act.astype(jnp.int32),
                                    nc1, nc2, m2n]
        return tuple(out)

    lax.while_loop(cond, body, tuple(state0))

    for k in range(GPW):
        wait_dma(k, rowa_v, sema)
        wait_dma(k, rowb_v, semb)
        pltpu.sync_copy(keep_v[k], keep_hbm.at[gs[k]])
        pltpu.sync_copy(ret_v[k], ret_hbm.at[gs[k]])


@functools.partial(jax.jit, static_argnums=(3, 4, 5))
def _sc_nms(sim_rows, scores_t, thr, bg, n, n_sl):
    mesh = plsc.VectorSubcoreMesh(core_axis_name="c", subcore_axis_name="s",
                                  num_cores=NC, num_subcores=NS)
    body = functools.partial(_nms_body, n, n_sl)
    return pl.kernel(
        body,
        out_type=[jax.ShapeDtypeStruct((bg, n), jnp.int32),
                  jax.ShapeDtypeStruct((bg, n), jnp.float32)],
        mesh=mesh,
        compiler_params=pltpu.CompilerParams(needs_layout_passes=False),
        scratch_types=(
            [pltpu.VMEM((n,), jnp.float32)] * GPW
            + [pltpu.VMEM((n,), jnp.float32)] * GPW
            + [pltpu.VMEM((n,), jnp.int32)] * GPW
            + [pltpu.VMEM((n,), jnp.float32)] * GPW
            + [pltpu.VMEM((n,), jnp.float32)] * GPW
            + [pltpu.VMEM((L,), jnp.float32)] * GPW
            + [pltpu.SemaphoreType.DMA] * GPW
            + [pltpu.SemaphoreType.DMA] * GPW
        ),
    )(sim_rows, scores_t, thr)


def kernel(similarity_matrix, scores, threshold):
    B, G, N, _ = similarity_matrix.shape
    bg = B * G
    assert bg == NW * GPW and N % L == 0
    sim_rows = similarity_matrix.reshape(bg * N, N)
    scores_t = jnp.transpose(scores, (0, 2, 1)).reshape(bg, N)
    thr = jnp.broadcast_to(threshold[:, None, None], (B, G, L)).reshape(bg, L)
    keep_flat, ret_flat = _sc_nms(sim_rows, scores_t, thr, bg, N, N // L)
    keep = keep_flat.reshape(B, G, N)
    ret = jnp.transpose(ret_flat.reshape(B, G, N), (0, 2, 1))
    return keep, ret
